# baseline (speedup 1.0000x reference)
import functools

import jax
import jax.numpy as jnp
import numpy as np
from jax import lax
from jax.experimental import pallas as pl
from jax.experimental.pallas import tpu as pltpu
from jax.experimental.pallas import tpu_sc as plsc

D_MODEL = 2048
SCALE = float(np.sqrt(np.float32(D_MODEL)))

NC = 2
NS = 16
L = 16
NW = NC * NS

B = 4 * 4096
BPW = B // NW
C = 8
NCHUNK = BPW // C
NG = 4
NO = 2
NROUND = NCHUNK // NG
NV = D_MODEL // L
UNROLL = 8

_mesh = plsc.VectorSubcoreMesh(core_axis_name="c", subcore_axis_name="s")


@functools.partial(
    pl.kernel,
    mesh=_mesh,
    out_type=jax.ShapeDtypeStruct((B, D_MODEL), jnp.float32),
    scratch_types=[
        pltpu.VMEM((BPW,), jnp.int32),
        pltpu.VMEM((C, D_MODEL), jnp.float32),
        pltpu.VMEM((C, D_MODEL), jnp.float32),
        pltpu.VMEM((C, D_MODEL), jnp.float32),
        pltpu.VMEM((C, D_MODEL), jnp.float32),
        pltpu.VMEM((C, D_MODEL), jnp.float32),
        pltpu.VMEM((C, D_MODEL), jnp.float32),
        pltpu.SemaphoreType.DMA,
        pltpu.SemaphoreType.DMA,
        pltpu.SemaphoreType.DMA,
        pltpu.SemaphoreType.DMA,
        pltpu.SemaphoreType.DMA,
        pltpu.SemaphoreType.DMA,
    ],
)
def _emb_lookup(table_hbm, idx_hbm, out_hbm, idx_v,
                gb0, gb1, gb2, gb3, ob0, ob1,
                gs0, gs1, gs2, gs3, ws0, ws1):
    gb = (gb0, gb1, gb2, gb3)
    ob = (ob0, ob1)
    gs = (gs0, gs1, gs2, gs3)
    ws = (ws0, ws1)

    wid = lax.axis_index("s") * NC + lax.axis_index("c")
    base = wid * BPW
    pltpu.sync_copy(idx_hbm.at[pl.ds(base, BPW)], idx_v)

    def start_gather(c, b):
        off = pl.multiple_of(c * C, 8)
        pltpu.async_copy(table_hbm.at[idx_v.at[pl.ds(off, C)]], gb[b], gs[b])

    def wait_gather(b):
        pltpu.make_async_copy(
            table_hbm.at[idx_v.at[pl.ds(0, C)]], gb[b], gs[b]).wait()

    def start_wb(c, o):
        off = pl.multiple_of(c * C, 8)
        pltpu.async_copy(ob[o], out_hbm.at[pl.ds(base + off, C)], ws[o])

    def wait_wb(o):
        pltpu.make_async_copy(ob[o], out_hbm.at[pl.ds(0, C)], ws[o]).wait()

    def scale(b, o):
        src = gb[b]
        dst = ob[o]
        for i in range(C):
            def inner(t, carry):
                for u in range(UNROLL):
                    sl = pl.ds(t * (UNROLL * L) + u * L, L)
                    dst[i, sl] = src[i, sl] * SCALE
                return carry
            lax.fori_loop(0, NV // UNROLL, inner, 0)

    def do_round(r, first, last):
        for k in range(NG):
            c = NG * r + k
            o = k % NO
            if not last or k == 0:
                start_gather(c + NG - 1, (k + NG - 1) % NG)
            wait_gather(k)
            if not (first and k < NO):
                wait_wb(o)
            scale(k, o)
            start_wb(c, o)

    start_gather(0, 0)
    start_gather(1, 1)
    start_gather(2, 2)
    do_round(0, True, False)
    lax.fori_loop(1, NROUND - 1,
                  lambda r, carry: (do_round(r, False, False), carry)[1], 0)
    do_round(NROUND - 1, False, True)
    wait_wb(0)
    wait_wb(1)


def kernel(x, W):
    idx = x.reshape(-1).astype(jnp.int32)
    out = _emb_lookup(W, idx)
    return out.reshape(x.shape[0], x.shape[1], D_MODEL)
